# baseline (speedup 1.0000x reference)
import functools

import numpy as np
import jax
import jax.numpy as jnp
from jax.experimental import pallas as pl

_N = 21
_HEADS = 4
_HC = 32
_F = _HEADS * _HC

_EDG_SRC = np.array([0, 0, 0, 0, 0, 1, 2, 3, 4, 5, 6, 7, 8, 9, 10, 11, 12, 13, 14, 15], dtype=np.int32)
_EDG_DST = np.array([13, 1, 4, 10, 7, 2, 3, 17, 5, 6, 18, 8, 9, 20, 11, 12, 19, 14, 15, 16], dtype=np.int32)


def _adj_mask_np():
    m = np.zeros((_N, _N), np.float32)
    m[_EDG_DST, _EDG_SRC] = 1.0
    m[np.arange(_N), np.arange(_N)] = 1.0
    return m


def _laplacian_np():
    a = np.zeros((_N, _N), np.float32)
    a[_EDG_SRC, _EDG_DST] = 1.0
    a[_EDG_DST, _EDG_SRC] = 1.0
    a += np.eye(_N, dtype=np.float32)
    a /= a.sum(axis=1, keepdims=True)
    deg = a.sum(-1)
    dm = np.diag(1.0 / np.sqrt(deg))
    return (np.eye(_N, dtype=np.float32) - dm @ a @ dm).astype(np.float32)


def _qstats_body(tgt_ref, feat_ref, q_ref, stats_ref):
    i = pl.program_id(0)
    t = tgt_ref[...]
    f = feat_ref[...]
    q = jax.lax.dot_general(
        t, f, (((2,), (2,)), ((0,), (0,))),
        preferred_element_type=jnp.float32)
    q_ref[...] = q
    s = jnp.sum(q, axis=(0, 2))
    ss = jnp.sum(q * q, axis=(0, 2))
    st = jnp.stack([s, ss], axis=0)

    @pl.when(i == 0)
    def _():
        stats_ref[...] = st

    @pl.when(i > 0)
    def _():
        stats_ref[...] = stats_ref[...] + st


def _graph_body(cnt, q_ref, stats_ref, gamma_ref, beta_ref, w_ref, asrc_ref,
                adst_ref, gbias_ref, w0_ref, w1_ref, cbias_ref, mask_ref,
                lap_ref, out_ref):
    q = q_ref[...]
    stats = stats_ref[...]
    mean = stats[0:1, :] / cnt
    var = stats[1:2, :] / cnt - mean * mean
    scale = gamma_ref[...] / jnp.sqrt(var + 1e-5)
    shift = beta_ref[...] - mean * scale
    qn = q * scale.reshape(1, _N, 1) + shift.reshape(1, _N, 1)
    qn = jnp.where(qn > 0, qn, 0.1 * qn)

    xw = jax.lax.dot_general(
        qn, w_ref[...], (((2,), (0,)), ((), ())),
        preferred_element_type=jnp.float32)

    asv = asrc_ref[...]
    adv = adst_ref[...]
    mask = mask_ref[...]
    outs = []
    for h in range(_HEADS):
        sl = slice(h * _HC, (h + 1) * _HC)
        xwh = xw[:, :, sl]
        ash = jnp.sum(xwh * asv[:, :, sl], axis=2)
        adh = jnp.sum(xwh * adv[:, :, sl], axis=2)
        e = adh[:, :, None] + ash[:, None, :]
        e = jnp.where(e > 0, e, 0.2 * e)
        e = jnp.where(mask[None] > 0, e, -1e30)
        emax = jnp.max(e, axis=2, keepdims=True)
        p = jnp.exp(e - emax) * mask[None]
        denom = jnp.sum(p, axis=2, keepdims=True) + 1e-16
        alpha = p / denom
        outs.append(jax.lax.dot_general(
            alpha, xwh, (((2,), (1,)), ((0,), (0,))),
            preferred_element_type=jnp.float32))
    x = jnp.concatenate(outs, axis=2) + gbias_ref[...]

    z0 = jax.lax.dot_general(
        x, w0_ref[...], (((2,), (0,)), ((), ())),
        preferred_element_type=jnp.float32)
    z1 = jax.lax.dot_general(
        x, w1_ref[...], (((2,), (0,)), ((), ())),
        preferred_element_type=jnp.float32)
    lb = jnp.broadcast_to(lap_ref[...][None], (z1.shape[0], _N, _N))
    z1l = jax.lax.dot_general(
        lb, z1, (((2,), (1,)), ((0,), (0,))),
        preferred_element_type=jnp.float32)
    out_ref[...] = z0 + z1l + cbias_ref[...]


def kernel(feature_s, target_s, bn_gamma, bn_beta, gat_W, gat_att_src,
           gat_att_dst, gat_bias, cheb_W, cheb_bias):
    B, C, H, W = feature_s.shape
    HW = H * W
    feat = feature_s.astype(jnp.bfloat16).reshape(B, C, HW)
    tgt = target_s.astype(jnp.bfloat16).reshape(B, _N, HW)

    BB1 = 16
    q, stats = pl.pallas_call(
        _qstats_body,
        grid=(B // BB1,),
        in_specs=[
            pl.BlockSpec((BB1, _N, HW), lambda i: (i, 0, 0)),
            pl.BlockSpec((BB1, C, HW), lambda i: (i, 0, 0)),
        ],
        out_specs=[
            pl.BlockSpec((BB1, _N, C), lambda i: (i, 0, 0)),
            pl.BlockSpec((2, _N), lambda i: (0, 0)),
        ],
        out_shape=[
            jax.ShapeDtypeStruct((B, _N, C), jnp.float32),
            jax.ShapeDtypeStruct((2, _N), jnp.float32),
        ],
    )(tgt, feat)

    BB2 = 32
    full = lambda shape: pl.BlockSpec(shape, lambda i: tuple(0 for _ in shape))
    out = pl.pallas_call(
        functools.partial(_graph_body, float(B * C)),
        grid=(B // BB2,),
        in_specs=[
            pl.BlockSpec((BB2, _N, C), lambda i: (i, 0, 0)),
            full((2, _N)),
            full((1, _N)),
            full((1, _N)),
            full((C, _F)),
            full((1, 1, _F)),
            full((1, 1, _F)),
            full((1, 1, _F)),
            full((_F, 3)),
            full((_F, 3)),
            full((1, 1, 3)),
            full((_N, _N)),
            full((_N, _N)),
        ],
        out_specs=pl.BlockSpec((BB2, _N, 3), lambda i: (i, 0, 0)),
        out_shape=jax.ShapeDtypeStruct((B, _N, 3), jnp.float32),
    )(q, stats,
      bn_gamma.reshape(1, _N), bn_beta.reshape(1, _N),
      gat_W,
      gat_att_src.reshape(1, 1, _F), gat_att_dst.reshape(1, 1, _F),
      gat_bias.reshape(1, 1, _F),
      cheb_W[0], cheb_W[1], cheb_bias.reshape(1, 1, 3),
      jnp.asarray(_adj_mask_np()), jnp.asarray(_laplacian_np()))
    return out


# speedup vs baseline: 1.6071x; 1.6071x over previous
import functools

import numpy as np
import jax
import jax.numpy as jnp
from jax.experimental import pallas as pl

_N = 21
_HEADS = 4
_HC = 32
_F = _HEADS * _HC

_EDG_SRC = np.array([0, 0, 0, 0, 0, 1, 2, 3, 4, 5, 6, 7, 8, 9, 10, 11, 12, 13, 14, 15], dtype=np.int32)
_EDG_DST = np.array([13, 1, 4, 10, 7, 2, 3, 17, 5, 6, 18, 8, 9, 20, 11, 12, 19, 14, 15, 16], dtype=np.int32)


def _adj_mask_np():
    m = np.zeros((_N, _N), np.float32)
    m[_EDG_DST, _EDG_SRC] = 1.0
    m[np.arange(_N), np.arange(_N)] = 1.0
    return m


def _laplacian_np():
    a = np.zeros((_N, _N), np.float32)
    a[_EDG_SRC, _EDG_DST] = 1.0
    a[_EDG_DST, _EDG_SRC] = 1.0
    a += np.eye(_N, dtype=np.float32)
    a /= a.sum(axis=1, keepdims=True)
    deg = a.sum(-1)
    dm = np.diag(1.0 / np.sqrt(deg))
    return (np.eye(_N, dtype=np.float32) - dm @ a @ dm).astype(np.float32)


def _transpose_cast_body(src_ref, dst_ref):
    dst_ref[...] = jnp.transpose(
        src_ref[...].astype(jnp.bfloat16), (2, 0, 1))


def _qstats_body(tgt_ref, feat_ref, q_ref, stats_ref):
    i = pl.program_id(0)
    t = tgt_ref[...].astype(jnp.bfloat16)
    f = feat_ref[...]
    q = jax.lax.dot_general(
        t, f, (((2,), (2,)), ((0,), (0,))),
        preferred_element_type=jnp.float32)
    q_ref[...] = q
    s = jnp.sum(q, axis=(0, 2))
    ss = jnp.sum(q * q, axis=(0, 2))
    st = jnp.stack([s, ss], axis=0)

    @pl.when(i == 0)
    def _():
        stats_ref[...] = st

    @pl.when(i > 0)
    def _():
        stats_ref[...] = stats_ref[...] + st


def _graph_body(cnt, q_ref, stats_ref, gamma_ref, beta_ref, w_ref, asrc_ref,
                adst_ref, gbias_ref, w0_ref, w1_ref, cbias_ref, mask_ref,
                lap_ref, out_ref):
    q = q_ref[...]
    stats = stats_ref[...]
    mean = stats[0:1, :] / cnt
    var = stats[1:2, :] / cnt - mean * mean
    scale = gamma_ref[...] / jnp.sqrt(var + 1e-5)
    shift = beta_ref[...] - mean * scale
    qn = q * scale.reshape(1, _N, 1) + shift.reshape(1, _N, 1)
    qn = jnp.where(qn > 0, qn, 0.1 * qn)

    xw = jax.lax.dot_general(
        qn, w_ref[...], (((2,), (0,)), ((), ())),
        preferred_element_type=jnp.float32)

    asv = asrc_ref[...]
    adv = adst_ref[...]
    mask = mask_ref[...]
    outs = []
    for h in range(_HEADS):
        sl = slice(h * _HC, (h + 1) * _HC)
        xwh = xw[:, :, sl]
        ash = jnp.sum(xwh * asv[:, :, sl], axis=2)
        adh = jnp.sum(xwh * adv[:, :, sl], axis=2)
        e = adh[:, :, None] + ash[:, None, :]
        e = jnp.where(e > 0, e, 0.2 * e)
        e = jnp.where(mask[None] > 0, e, -1e30)
        emax = jnp.max(e, axis=2, keepdims=True)
        p = jnp.exp(e - emax) * mask[None]
        denom = jnp.sum(p, axis=2, keepdims=True) + 1e-16
        alpha = p / denom
        outs.append(jax.lax.dot_general(
            alpha, xwh, (((2,), (1,)), ((0,), (0,))),
            preferred_element_type=jnp.float32))
    x = jnp.concatenate(outs, axis=2) + gbias_ref[...]

    z0 = jax.lax.dot_general(
        x, w0_ref[...], (((2,), (0,)), ((), ())),
        preferred_element_type=jnp.float32)
    z1 = jax.lax.dot_general(
        x, w1_ref[...], (((2,), (0,)), ((), ())),
        preferred_element_type=jnp.float32)
    lb = jnp.broadcast_to(lap_ref[...][None], (z1.shape[0], _N, _N))
    z1l = jax.lax.dot_general(
        lb, z1, (((2,), (1,)), ((0,), (0,))),
        preferred_element_type=jnp.float32)
    out_ref[...] = z0 + z1l + cbias_ref[...]


def kernel(feature_s, target_s, bn_gamma, bn_beta, gat_W, gat_att_src,
           gat_att_dst, gat_bias, cheb_W, cheb_bias):
    B, C, H, W = feature_s.shape
    HW = H * W
    feat_t = feature_s.transpose(1, 2, 3, 0).reshape(C, HW, B)
    HB, BBK = 128, 128
    feat = pl.pallas_call(
        _transpose_cast_body,
        grid=(B // BBK, HW // HB),
        in_specs=[pl.BlockSpec((C, HB, BBK), lambda i, j: (0, j, i))],
        out_specs=pl.BlockSpec((BBK, C, HB), lambda i, j: (i, 0, j)),
        out_shape=jax.ShapeDtypeStruct((B, C, HW), jnp.bfloat16),
    )(feat_t)
    tgt = target_s.reshape(B, _N, HW)

    BB1 = 16
    q, stats = pl.pallas_call(
        _qstats_body,
        grid=(B // BB1,),
        in_specs=[
            pl.BlockSpec((BB1, _N, HW), lambda i: (i, 0, 0)),
            pl.BlockSpec((BB1, C, HW), lambda i: (i, 0, 0)),
        ],
        out_specs=[
            pl.BlockSpec((BB1, _N, C), lambda i: (i, 0, 0)),
            pl.BlockSpec((2, _N), lambda i: (0, 0)),
        ],
        out_shape=[
            jax.ShapeDtypeStruct((B, _N, C), jnp.float32),
            jax.ShapeDtypeStruct((2, _N), jnp.float32),
        ],
    )(tgt, feat)

    BB2 = 32
    full = lambda shape: pl.BlockSpec(shape, lambda i: tuple(0 for _ in shape))
    out = pl.pallas_call(
        functools.partial(_graph_body, float(B * C)),
        grid=(B // BB2,),
        in_specs=[
            pl.BlockSpec((BB2, _N, C), lambda i: (i, 0, 0)),
            full((2, _N)),
            full((1, _N)),
            full((1, _N)),
            full((C, _F)),
            full((1, 1, _F)),
            full((1, 1, _F)),
            full((1, 1, _F)),
            full((_F, 3)),
            full((_F, 3)),
            full((1, 1, 3)),
            full((_N, _N)),
            full((_N, _N)),
        ],
        out_specs=pl.BlockSpec((BB2, _N, 3), lambda i: (i, 0, 0)),
        out_shape=jax.ShapeDtypeStruct((B, _N, 3), jnp.float32),
    )(q, stats,
      bn_gamma.reshape(1, _N), bn_beta.reshape(1, _N),
      gat_W,
      gat_att_src.reshape(1, 1, _F), gat_att_dst.reshape(1, 1, _F),
      gat_bias.reshape(1, 1, _F),
      cheb_W[0], cheb_W[1], cheb_bias.reshape(1, 1, 3),
      jnp.asarray(_adj_mask_np()), jnp.asarray(_laplacian_np()))
    return out


# speedup vs baseline: 2.5102x; 1.5620x over previous
import functools

import numpy as np
import jax
import jax.numpy as jnp
from jax.experimental import pallas as pl
from jax.experimental.pallas import tpu as pltpu

_N = 21
_HEADS = 4
_HC = 32
_F = _HEADS * _HC

_EDG_SRC = np.array([0, 0, 0, 0, 0, 1, 2, 3, 4, 5, 6, 7, 8, 9, 10, 11, 12, 13, 14, 15], dtype=np.int32)
_EDG_DST = np.array([13, 1, 4, 10, 7, 2, 3, 17, 5, 6, 18, 8, 9, 20, 11, 12, 19, 14, 15, 16], dtype=np.int32)


def _adj_mask_np():
    m = np.zeros((_N, _N), np.float32)
    m[_EDG_DST, _EDG_SRC] = 1.0
    m[np.arange(_N), np.arange(_N)] = 1.0
    return m


def _laplacian_np():
    a = np.zeros((_N, _N), np.float32)
    a[_EDG_SRC, _EDG_DST] = 1.0
    a[_EDG_DST, _EDG_SRC] = 1.0
    a += np.eye(_N, dtype=np.float32)
    a /= a.sum(axis=1, keepdims=True)
    deg = a.sum(-1)
    dm = np.diag(1.0 / np.sqrt(deg))
    return (np.eye(_N, dtype=np.float32) - dm @ a @ dm).astype(np.float32)


def _qstats_fused_body(nh, tgt_ref, feat_ref, q_ref, stats_ref, acc_ref):
    i = pl.program_id(0)
    j = pl.program_id(1)
    fb = feat_ref[...].astype(jnp.bfloat16)
    tb = tgt_ref[...].astype(jnp.bfloat16)
    partial = jax.lax.dot_general(
        tb, fb, (((1,), (1,)), ((2,), (2,))),
        preferred_element_type=jnp.float32)

    @pl.when(j == 0)
    def _():
        acc_ref[...] = partial

    @pl.when(j > 0)
    def _():
        acc_ref[...] = acc_ref[...] + partial

    @pl.when(j == nh - 1)
    def _():
        q = acc_ref[...]
        q_ref[...] = q.astype(jnp.bfloat16)
        s = jnp.sum(q, axis=(0, 2))
        ss = jnp.sum(q * q, axis=(0, 2))
        st = jnp.stack([s, ss], axis=0)

        @pl.when(i == 0)
        def _():
            stats_ref[...] = st

        @pl.when(i > 0)
        def _():
            stats_ref[...] = stats_ref[...] + st


def _graph_body(cnt, q_ref, stats_ref, gamma_ref, beta_ref, w_ref, asrc_ref,
                adst_ref, gbias_ref, w0_ref, w1_ref, cbias_ref, mask_ref,
                lap_ref, out_ref):
    q = q_ref[...].astype(jnp.float32)
    stats = stats_ref[...]
    mean = stats[0:1, :] / cnt
    var = stats[1:2, :] / cnt - mean * mean
    scale = gamma_ref[...] / jnp.sqrt(var + 1e-5)
    shift = beta_ref[...] - mean * scale
    qn = q * scale.reshape(1, _N, 1) + shift.reshape(1, _N, 1)
    qn = jnp.where(qn > 0, qn, 0.1 * qn)

    xw = jax.lax.dot_general(
        qn, w_ref[...], (((2,), (0,)), ((), ())),
        preferred_element_type=jnp.float32)

    asv = asrc_ref[...]
    adv = adst_ref[...]
    mask = mask_ref[...]
    outs = []
    for h in range(_HEADS):
        sl = slice(h * _HC, (h + 1) * _HC)
        xwh = xw[:, :, sl]
        ash = jnp.sum(xwh * asv[:, :, sl], axis=2)
        adh = jnp.sum(xwh * adv[:, :, sl], axis=2)
        e = adh[:, :, None] + ash[:, None, :]
        e = jnp.where(e > 0, e, 0.2 * e)
        e = jnp.where(mask[None] > 0, e, -1e30)
        emax = jnp.max(e, axis=2, keepdims=True)
        p = jnp.exp(e - emax) * mask[None]
        denom = jnp.sum(p, axis=2, keepdims=True) + 1e-16
        alpha = p / denom
        outs.append(jax.lax.dot_general(
            alpha, xwh, (((2,), (1,)), ((0,), (0,))),
            preferred_element_type=jnp.float32))
    x = jnp.concatenate(outs, axis=2) + gbias_ref[...]

    z0 = jax.lax.dot_general(
        x, w0_ref[...], (((2,), (0,)), ((), ())),
        preferred_element_type=jnp.float32)
    z1 = jax.lax.dot_general(
        x, w1_ref[...], (((2,), (0,)), ((), ())),
        preferred_element_type=jnp.float32)
    lb = jnp.broadcast_to(lap_ref[...][None], (z1.shape[0], _N, _N))
    z1l = jax.lax.dot_general(
        lb, z1, (((2,), (1,)), ((0,), (0,))),
        preferred_element_type=jnp.float32)
    out_ref[...] = z0 + z1l + cbias_ref[...]


def kernel(feature_s, target_s, bn_gamma, bn_beta, gat_W, gat_att_src,
           gat_att_dst, gat_bias, cheb_W, cheb_bias):
    B, C, H, W = feature_s.shape
    HW = H * W
    feat_t = feature_s.transpose(1, 2, 3, 0).reshape(C, HW, B)
    tgt_t = target_s.transpose(1, 2, 3, 0).reshape(_N, HW, B)

    HB, BBK = 128, 128
    nh = HW // HB
    q, stats = pl.pallas_call(
        functools.partial(_qstats_fused_body, nh),
        grid=(B // BBK, nh),
        in_specs=[
            pl.BlockSpec((_N, HB, BBK), lambda i, j: (0, j, i)),
            pl.BlockSpec((C, HB, BBK), lambda i, j: (0, j, i)),
        ],
        out_specs=[
            pl.BlockSpec((BBK, _N, C), lambda i, j: (i, 0, 0)),
            pl.BlockSpec((2, _N), lambda i, j: (0, 0)),
        ],
        out_shape=[
            jax.ShapeDtypeStruct((B, _N, C), jnp.bfloat16),
            jax.ShapeDtypeStruct((2, _N), jnp.float32),
        ],
        scratch_shapes=[pltpu.VMEM((BBK, _N, C), jnp.float32)],
    )(tgt_t, feat_t)

    BB2 = 32
    full = lambda shape: pl.BlockSpec(shape, lambda i: tuple(0 for _ in shape))
    out = pl.pallas_call(
        functools.partial(_graph_body, float(B * C)),
        grid=(B // BB2,),
        in_specs=[
            pl.BlockSpec((BB2, _N, C), lambda i: (i, 0, 0)),
            full((2, _N)),
            full((1, _N)),
            full((1, _N)),
            full((C, _F)),
            full((1, 1, _F)),
            full((1, 1, _F)),
            full((1, 1, _F)),
            full((_F, 3)),
            full((_F, 3)),
            full((1, 1, 3)),
            full((_N, _N)),
            full((_N, _N)),
        ],
        out_specs=pl.BlockSpec((BB2, _N, 3), lambda i: (i, 0, 0)),
        out_shape=jax.ShapeDtypeStruct((B, _N, 3), jnp.float32),
    )(q, stats,
      bn_gamma.reshape(1, _N), bn_beta.reshape(1, _N),
      gat_W,
      gat_att_src.reshape(1, 1, _F), gat_att_dst.reshape(1, 1, _F),
      gat_bias.reshape(1, 1, _F),
      cheb_W[0], cheb_W[1], cheb_bias.reshape(1, 1, 3),
      jnp.asarray(_adj_mask_np()), jnp.asarray(_laplacian_np()))
    return out
